# Initial kernel scaffold; baseline (speedup 1.0000x reference)
import jax
import jax.numpy as jnp
from jax.experimental import pallas as pl


def kernel(x, edge_index, edge_weight, We, be, W1, b1, W2, b2):
    raise NotImplementedError("write your pallas kernel here")


# speedup vs baseline: 1.5305x; 1.5305x over previous
import jax
import jax.numpy as jnp
from jax import lax
from jax.experimental import pallas as pl
from jax.experimental.pallas import tpu as pltpu
from jax.experimental.pallas import tpu_sc as plsc

HIDDEN = 128
N_NODES = 10000
N_EDGES = 320000

NC = 2
NS = 16
NW = NC * NS
E_PER_TILE = N_EDGES // NW
CHUNK = 80
NCHUNKS = E_PER_TILE // CHUNK
N_PAD = 10240
ROWS_PER_TILE = N_PAD // NS


SDW = 16


def _sc_body(x_hbm, src_hbm, dst_hbm, sd_hbm, z128_hbm, zsd_hbm,
             agg_out, sd_out,
             src_v, dst_v, sd_v, rows_v, acc_sh, sd_sh, sem):
  c = lax.axis_index("c")
  s = lax.axis_index("s")
  wid = c * NS + s
  ebase = wid * E_PER_TILE

  rbase = s * ROWS_PER_TILE
  pltpu.sync_copy(z128_hbm.at[pl.ds(rbase, ROWS_PER_TILE)],
                  acc_sh.at[pl.ds(rbase, ROWS_PER_TILE)])
  pltpu.sync_copy(zsd_hbm.at[pl.ds(rbase, ROWS_PER_TILE)],
                  sd_sh.at[pl.ds(rbase, ROWS_PER_TILE)])
  plsc.subcore_barrier()

  def chunk_body(i, carry):
    off = ebase + i * CHUNK
    pltpu.sync_copy(src_hbm.at[pl.ds(off, CHUNK)], src_v)
    pltpu.sync_copy(dst_hbm.at[pl.ds(off, CHUNK)], dst_v)
    pltpu.sync_copy(sd_hbm.at[pl.ds(off, CHUNK)], sd_v)
    pltpu.async_copy(x_hbm.at[src_v], rows_v, sem).wait()
    pltpu.sync_copy(rows_v, acc_sh.at[dst_v], add=True)
    pltpu.sync_copy(sd_v, sd_sh.at[dst_v], add=True)
    return carry

  lax.fori_loop(0, NCHUNKS, chunk_body, 0)
  plsc.subcore_barrier()

  obase = c * N_PAD + rbase
  pltpu.sync_copy(acc_sh.at[pl.ds(rbase, ROWS_PER_TILE)],
                  agg_out.at[pl.ds(obase, ROWS_PER_TILE)])
  pltpu.sync_copy(sd_sh.at[pl.ds(rbase, ROWS_PER_TILE)],
                  sd_out.at[pl.ds(obase, ROWS_PER_TILE)])


_sc_aggregate = pl.kernel(
    _sc_body,
    out_type=(
        jax.ShapeDtypeStruct((NC * N_PAD, HIDDEN), jnp.float32),
        jax.ShapeDtypeStruct((NC * N_PAD, SDW), jnp.float32),
    ),
    mesh=plsc.VectorSubcoreMesh(core_axis_name="c", subcore_axis_name="s",
                                num_cores=NC, num_subcores=NS),
    scratch_types=[
        pltpu.VMEM((CHUNK,), jnp.int32),
        pltpu.VMEM((CHUNK,), jnp.int32),
        pltpu.VMEM((CHUNK, SDW), jnp.float32),
        pltpu.VMEM((CHUNK, HIDDEN), jnp.float32),
        pltpu.VMEM_SHARED((N_PAD, HIDDEN), jnp.float32),
        pltpu.VMEM_SHARED((N_PAD, SDW), jnp.float32),
        pltpu.SemaphoreType.DMA,
    ],
    compiler_params=pltpu.CompilerParams(use_tc_tiling_on_sc=False),
)


ROW_BLK = 1000


def _mlp_body(p0, p1, sd0, sd1, x, We, be, W1, b1, W2, b2, o):
  sd = sd0[...] + sd1[...]
  pre = (p0[...] + p1[...] + x[...]
         + sd[:, 0:1] * We[...] + sd[:, 1:2] * be[...])
  h = jnp.maximum(
      jnp.dot(pre, W1[...], preferred_element_type=jnp.float32) + b1[...], 0.0)
  o[...] = jnp.dot(h, W2[...], preferred_element_type=jnp.float32) + b2[...]


def _mlp_call(p0, p1, sd0, sd1, x, We, be, W1, b1, W2, b2):
  grid = (N_NODES // ROW_BLK,)
  row = lambda i: (i, 0)
  fix = lambda i: (0, 0)
  return pl.pallas_call(
      _mlp_body,
      grid=grid,
      in_specs=[
          pl.BlockSpec((ROW_BLK, HIDDEN), row),
          pl.BlockSpec((ROW_BLK, HIDDEN), row),
          pl.BlockSpec((ROW_BLK, SDW), row),
          pl.BlockSpec((ROW_BLK, SDW), row),
          pl.BlockSpec((ROW_BLK, HIDDEN), row),
          pl.BlockSpec((1, HIDDEN), fix),
          pl.BlockSpec((1, HIDDEN), fix),
          pl.BlockSpec((HIDDEN, HIDDEN), fix),
          pl.BlockSpec((1, HIDDEN), fix),
          pl.BlockSpec((HIDDEN, HIDDEN), fix),
          pl.BlockSpec((1, HIDDEN), fix),
      ],
      out_specs=pl.BlockSpec((ROW_BLK, HIDDEN), row),
      out_shape=jax.ShapeDtypeStruct((N_NODES, HIDDEN), jnp.float32),
  )(p0, p1, sd0, sd1, x, We, be, W1, b1, W2, b2)


def kernel(x, edge_index, edge_weight, We, be, W1, b1, W2, b2):
  src = edge_index[0].astype(jnp.int32)
  dst = edge_index[1].astype(jnp.int32)
  ew = edge_weight.astype(jnp.float32)
  sd = (jnp.zeros((N_EDGES, SDW), jnp.float32)
        .at[:, 0].set(ew).at[:, 1].set(1.0))
  z128 = jnp.zeros((N_PAD, HIDDEN), jnp.float32)
  zsd = jnp.zeros((N_PAD, SDW), jnp.float32)
  agg, sdp = _sc_aggregate(x, src, dst, sd, z128, zsd)
  return _mlp_call(agg[:N_NODES], agg[N_PAD:N_PAD + N_NODES],
                   sdp[:N_NODES], sdp[N_PAD:N_PAD + N_NODES],
                   x, We, be.reshape(1, HIDDEN), W1, b1.reshape(1, HIDDEN),
                   W2, b2.reshape(1, HIDDEN))
